# Initial kernel scaffold; baseline (speedup 1.0000x reference)
import jax
import jax.numpy as jnp
from jax.experimental import pallas as pl


def kernel(x, W_ih, W_hh, b_ih, b_hh, W_out, b_out):
    raise NotImplementedError("write your pallas kernel here")


# speedup vs baseline: 22.0977x; 22.0977x over previous
import jax
import jax.numpy as jnp
from jax.experimental import pallas as pl
from jax.experimental.pallas import tpu as pltpu

INPUT_SIZE = 16
HIDDEN = 512
G4 = 4 * HIDDEN
WINDOW = 168
WARMUP = 1344
WEEK = 672
THRESH = 3.0
MIN_REQ = 84
MAD_K = 1.4826
BIG = 1e30
CH = 8


def _sigmoid(x):
    return jax.nn.sigmoid(x)


def _fwd_kernel(x_ref, wx_ref, whh_ref, v_ref, wout_ref, bout_ref,
                preds_ref, hout_ref, cout_ref,
                h_s, c_s, pp_s, wrow_s, wcol_s):
    pid = pl.program_id(0)
    n_chunks = pl.num_programs(0)
    tbase = pid * CH
    f32 = jnp.float32

    @pl.when(pid == 0)
    def _init():
        h_s[...] = jnp.zeros_like(h_s)
        c_s[...] = jnp.zeros_like(c_s)
        pp_s[...] = jnp.zeros_like(pp_s)
        wrow_s[...] = jnp.full_like(wrow_s, BIG)
        wcol_s[...] = jnp.full_like(wcol_s, BIG)

    x_blk = x_ref[0]
    wx = wx_ref[...]
    whh = whh_ref[...]
    vm = v_ref[...]
    wout = wout_ref[...]
    bo = bout_ref[0]

    B = x_blk.shape[1]
    onec = jnp.ones((B, 1), f32)
    zeroc = jnp.zeros((B, 4), f32)

    def lstm_step(xt, h, c, ppcol, upf, bf, af):
        delta = (ppcol - xt[:, 0:1]) * upf
        u = jnp.concatenate([delta, bf * onec, af * onec, onec], axis=1)
        u = jnp.concatenate([u, zeroc], axis=1)
        gates = (jnp.dot(xt, wx, preferred_element_type=f32)
                 + jnp.dot(h, whh, preferred_element_type=f32)
                 + jnp.dot(u, vm, preferred_element_type=f32))
        i_g = gates[:, 0:HIDDEN]
        f_g = gates[:, HIDDEN:2 * HIDDEN]
        g_g = gates[:, 2 * HIDDEN:3 * HIDDEN]
        o_g = gates[:, 3 * HIDDEN:4 * HIDDEN]
        c_n = _sigmoid(f_g) * c + _sigmoid(i_g) * jnp.tanh(g_g)
        h_n = _sigmoid(o_g) * jnp.tanh(c_n)
        outmm = jnp.dot(h_n, wout, preferred_element_type=f32)
        pred = outmm[:, 0:1] + bo
        return h_n, c_n, pred

    h = h_s[...]
    c = c_s[...]
    ppcol = pp_s[...]

    @pl.when(pid < WARMUP // CH)
    def _warm():
        hh, cc, pp = h, c, ppcol
        upf = jnp.where(tbase >= WEEK, 1.0, 0.0).astype(f32)
        cols = []
        for r in range(CH):
            xt = x_blk[r]
            hh, cc, pp = lstm_step(xt, hh, cc, pp, upf, upf, 0.0)
            cols.append(pp)
        preds_ref[0] = jnp.concatenate(cols, axis=1)
        h_s[...] = hh
        c_s[...] = cc
        pp_s[...] = pp

    @pl.when(pid >= WARMUP // CH)
    def _zmode():
        hh, cc, pp = h, c, ppcol
        wrow = wrow_s[...]
        wcol = wcol_s[...]
        lane_i = jax.lax.broadcasted_iota(jnp.int32, (1, 256), 1)
        sub_i = jax.lax.broadcasted_iota(jnp.int32, (256, 1), 0)
        tb_mod = jax.lax.rem(tbase - WARMUP, WINDOW)
        pp0 = pp[0, 0]
        cols = []
        for r in range(CH):
            t = tbase + r
            xt = x_blk[r]
            resid = xt[0, 0] - pp0
            p = tb_mod + r
            wrow = jnp.where(lane_i == p, resid, wrow)
            wcol = jnp.where(sub_i == p, resid, wcol)
            nv = t - (WARMUP - 1)
            m = jnp.minimum(nv, WINDOW)
            kf = ((m - 1) // 2).astype(f32)
            wr = wrow[:, :WINDOW]
            wc = wcol[:WINDOW, :]
            cnt = jnp.sum(jnp.where(wr < wc, 1.0, 0.0), axis=1, keepdims=True)
            med = jnp.max(jnp.where(cnt <= kf, wc, -BIG))
            dr = jnp.abs(wr - med)
            dc = jnp.abs(wc - med)
            cnt2 = jnp.sum(jnp.where(dr < dc, 1.0, 0.0), axis=1, keepdims=True)
            mad = jnp.max(jnp.where(cnt2 <= kf, dc, -BIG))
            valid = (nv >= MIN_REQ) & (mad != 0.0)
            denom = jnp.where(valid, mad * MAD_K, 1.0)
            z = jnp.where(valid, (resid - med) / denom, 0.0)
            anom = valid & (jnp.abs(z) > THRESH)
            af = jnp.where(anom, 1.0, 0.0).astype(f32)
            hh, cc, pp = lstm_step(xt, hh, cc, pp, af, af, af)
            pp0 = pp[0, 0]
            cols.append(pp)
        preds_ref[0] = jnp.concatenate(cols, axis=1)
        h_s[...] = hh
        c_s[...] = cc
        pp_s[...] = pp
        wrow_s[...] = wrow
        wcol_s[...] = wcol

    @pl.when(pid == n_chunks - 1)
    def _final():
        hout_ref[...] = h_s[...]
        cout_ref[...] = c_s[...]


def kernel(x, W_ih, W_hh, b_ih, b_hh, W_out, b_out):
    B, T, F = x.shape
    f32 = jnp.float32
    nc = T // CH
    xr = jnp.transpose(x, (1, 0, 2)).reshape(nc, CH, B, F)
    wx = jnp.transpose(W_ih[:, :F])
    whh = jnp.transpose(W_hh)
    bias = b_ih + b_hh
    vm = jnp.stack([W_ih[:, 0], W_ih[:, F], W_ih[:, F + 1], bias,
                    jnp.zeros_like(bias), jnp.zeros_like(bias),
                    jnp.zeros_like(bias), jnp.zeros_like(bias)], axis=0)
    wout = jnp.zeros((HIDDEN, 128), f32).at[:, 0].set(W_out[0])
    bo = b_out.astype(f32)

    preds, h, c = pl.pallas_call(
        _fwd_kernel,
        grid=(nc,),
        in_specs=[
            pl.BlockSpec((1, CH, B, F), lambda i: (i, 0, 0, 0)),
            pl.BlockSpec((F, G4), lambda i: (0, 0)),
            pl.BlockSpec((HIDDEN, G4), lambda i: (0, 0)),
            pl.BlockSpec((8, G4), lambda i: (0, 0)),
            pl.BlockSpec((HIDDEN, 128), lambda i: (0, 0)),
            pl.BlockSpec(memory_space=pltpu.SMEM),
        ],
        out_specs=[
            pl.BlockSpec((1, B, CH), lambda i: (i, 0, 0)),
            pl.BlockSpec((B, HIDDEN), lambda i: (0, 0)),
            pl.BlockSpec((B, HIDDEN), lambda i: (0, 0)),
        ],
        out_shape=[
            jax.ShapeDtypeStruct((nc, B, CH), f32),
            jax.ShapeDtypeStruct((B, HIDDEN), f32),
            jax.ShapeDtypeStruct((B, HIDDEN), f32),
        ],
        scratch_shapes=[
            pltpu.VMEM((B, HIDDEN), f32),
            pltpu.VMEM((B, HIDDEN), f32),
            pltpu.VMEM((B, 1), f32),
            pltpu.VMEM((1, 256), f32),
            pltpu.VMEM((256, 1), f32),
        ],
        compiler_params=pltpu.CompilerParams(
            dimension_semantics=("arbitrary",),
        ),
        name="alt_forecast_lstm",
    )(xr, wx, whh, vm, wout, bo)

    outputs = jnp.transpose(preds, (1, 0, 2)).reshape(B, T, 1)
    return outputs, h, c


# speedup vs baseline: 25.4239x; 1.1505x over previous
import jax
import jax.numpy as jnp
from jax.experimental import pallas as pl
from jax.experimental.pallas import tpu as pltpu

INPUT_SIZE = 16
HIDDEN = 512
G4 = 4 * HIDDEN
WINDOW = 168
WARMUP = 1344
WEEK = 672
THRESH = 3.0
MIN_REQ = 84
MAD_K = 1.4826
BIG = 1e30
CH = 8


def _fwd_kernel(x_ref, w2_ref, whh_ref, wout_ref, bout_ref,
                preds_ref, hout_ref, cout_ref,
                h_s, c_s, pp_s, wrow_s, wcol_s):
    pid = pl.program_id(0)
    n_chunks = pl.num_programs(0)
    tbase = pid * CH
    f32 = jnp.float32
    first = pid == 0

    x_blk = x_ref[0]
    w2 = w2_ref[...]
    whh = whh_ref[...]
    wout = wout_ref[...]
    bo = bout_ref[0]

    B = x_blk.shape[1]
    onec = jnp.ones((B, 1), f32)
    zc = jnp.zeros((B, 4), f32)

    def lstm_step(xt, h, c, ppcol, upf, bf, af):
        delta = (ppcol - xt[:, 0:1]) * upf
        xu = jnp.concatenate(
            [xt, delta, bf * onec, af * onec, onec, zc], axis=1)
        gates = (jnp.dot(xu, w2, preferred_element_type=f32)
                 + jnp.dot(h, whh, preferred_element_type=f32))
        i_g = gates[:, 0:HIDDEN]
        f_g = gates[:, HIDDEN:2 * HIDDEN]
        g_g = gates[:, 2 * HIDDEN:3 * HIDDEN]
        o_g = gates[:, 3 * HIDDEN:4 * HIDDEN]
        c_n = jax.nn.sigmoid(f_g) * c + jax.nn.sigmoid(i_g) * jnp.tanh(g_g)
        h_n = jax.nn.sigmoid(o_g) * jnp.tanh(c_n)
        outmm = jnp.dot(h_n, wout, preferred_element_type=f32)
        pred = outmm[:, 0:1] + bo
        return h_n, c_n, pred

    h = jnp.where(first, 0.0, h_s[...])
    c = jnp.where(first, 0.0, c_s[...])
    ppcol = jnp.where(first, 0.0, pp_s[...])
    one11 = jnp.ones((1, 1), f32)
    zero11 = jnp.zeros((1, 1), f32)

    @pl.when(pid < WARMUP // CH)
    def _warm():
        hh, cc, pp = h, c, ppcol
        upf = jnp.where(tbase >= WEEK, one11, zero11)
        cols = []
        for r in range(CH):
            hh, cc, pp = lstm_step(x_blk[r], hh, cc, pp, upf, upf, zero11)
            cols.append(pp)
        preds_ref[0] = jnp.concatenate(cols, axis=1)
        h_s[...] = hh
        c_s[...] = cc
        pp_s[...] = pp

    @pl.when(pid >= WARMUP // CH)
    def _zmode():
        hh, cc, pp = h, c, ppcol
        wrow = wrow_s[...]
        wcol = wcol_s[...]
        lane_i = jax.lax.broadcasted_iota(jnp.int32, (1, 256), 1)
        sub_i = jax.lax.broadcasted_iota(jnp.int32, (256, 1), 0)
        lane_w = lane_i[:, :WINDOW]
        sub_w = sub_i[:WINDOW, :]
        tb_mod = jax.lax.rem(tbase - WARMUP, WINDOW)
        pp0 = pp[0:1, 0:1]
        cols = []
        for r in range(CH):
            t = tbase + r
            xt = x_blk[r]
            resid = xt[0:1, 0:1] - pp0
            p = tb_mod + r
            wrow = jnp.where(lane_i == p, resid, wrow)
            wcol = jnp.where(sub_i == p, resid, wcol)
            nv = t - (WARMUP - 1)
            m = jnp.minimum(nv, WINDOW)
            kf = ((m - 1) // 2).astype(f32)
            wr = jnp.where(lane_w < m, wrow[:, :WINDOW], BIG)
            wc = jnp.where(sub_w < m, wcol[:WINDOW, :], BIG)
            cnt = jnp.sum(jnp.where(wc < wr, 1.0, 0.0), axis=0, keepdims=True)
            med = jnp.max(jnp.where(cnt <= kf, wr, -BIG), axis=1, keepdims=True)
            dr = jnp.abs(wr - med)
            dc = jnp.abs(wc - med)
            cnt2 = jnp.sum(jnp.where(dc < dr, 1.0, 0.0), axis=0, keepdims=True)
            mad = jnp.max(jnp.where(cnt2 <= kf, dr, -BIG), axis=1, keepdims=True)
            valid = (nv >= MIN_REQ) & (mad != 0.0)
            denom = jnp.where(valid, mad * MAD_K, 1.0)
            z = jnp.where(valid, (resid - med) / denom, 0.0)
            anom = valid & (jnp.abs(z) > THRESH)
            af = jnp.where(anom, one11, zero11)
            hh, cc, pp = lstm_step(xt, hh, cc, pp, af, af, af)
            pp0 = pp[0:1, 0:1]
            cols.append(pp)
        preds_ref[0] = jnp.concatenate(cols, axis=1)
        h_s[...] = hh
        c_s[...] = cc
        pp_s[...] = pp
        wrow_s[...] = wrow
        wcol_s[...] = wcol

    @pl.when(pid == n_chunks - 1)
    def _final():
        hout_ref[...] = h_s[...]
        cout_ref[...] = c_s[...]


def kernel(x, W_ih, W_hh, b_ih, b_hh, W_out, b_out):
    B, T, F = x.shape
    f32 = jnp.float32
    nc = T // CH
    xr = jnp.transpose(x, (1, 0, 2)).reshape(nc, CH, B, F)
    bias = b_ih + b_hh
    zrow = jnp.zeros_like(bias)
    w2 = jnp.stack(
        [W_ih[:, j] for j in range(F)]
        + [W_ih[:, 0], W_ih[:, F], W_ih[:, F + 1], bias, zrow, zrow, zrow, zrow],
        axis=0)
    whh = jnp.transpose(W_hh)
    wout = jnp.zeros((HIDDEN, 128), f32).at[:, 0].set(W_out[0])
    bo = b_out.astype(f32)

    preds, h, c = pl.pallas_call(
        _fwd_kernel,
        grid=(nc,),
        in_specs=[
            pl.BlockSpec((1, CH, B, F), lambda i: (i, 0, 0, 0)),
            pl.BlockSpec((24, G4), lambda i: (0, 0)),
            pl.BlockSpec((HIDDEN, G4), lambda i: (0, 0)),
            pl.BlockSpec((HIDDEN, 128), lambda i: (0, 0)),
            pl.BlockSpec(memory_space=pltpu.SMEM),
        ],
        out_specs=[
            pl.BlockSpec((1, B, CH), lambda i: (i, 0, 0)),
            pl.BlockSpec((B, HIDDEN), lambda i: (0, 0)),
            pl.BlockSpec((B, HIDDEN), lambda i: (0, 0)),
        ],
        out_shape=[
            jax.ShapeDtypeStruct((nc, B, CH), f32),
            jax.ShapeDtypeStruct((B, HIDDEN), f32),
            jax.ShapeDtypeStruct((B, HIDDEN), f32),
        ],
        scratch_shapes=[
            pltpu.VMEM((B, HIDDEN), f32),
            pltpu.VMEM((B, HIDDEN), f32),
            pltpu.VMEM((B, 1), f32),
            pltpu.VMEM((1, 256), f32),
            pltpu.VMEM((256, 1), f32),
        ],
        compiler_params=pltpu.CompilerParams(
            dimension_semantics=("arbitrary",),
        ),
        name="alt_forecast_lstm",
    )(xr, w2, whh, wout, bo)

    outputs = jnp.transpose(preds, (1, 0, 2)).reshape(B, T, 1)
    return outputs, h, c


# speedup vs baseline: 29.4352x; 1.1578x over previous
import jax
import jax.numpy as jnp
from jax.experimental import pallas as pl
from jax.experimental.pallas import tpu as pltpu

INPUT_SIZE = 16
HIDDEN = 512
G4 = 4 * HIDDEN
WINDOW = 168
WARMUP = 1344
WEEK = 672
THRESH = 3.0
MIN_REQ = 84
MAD_K = 1.4826
BIG = 1e30
CH = 16


def _fwd_kernel(x_ref, w2_ref, whh_ref, wout_ref, bout_ref,
                preds_ref, hout_ref, cout_ref,
                h_s, c_s, pp_s, wrow_s, wcol_s):
    pid = pl.program_id(0)
    n_chunks = pl.num_programs(0)
    tbase = pid * CH
    f32 = jnp.float32
    first = pid == 0

    x_blk = x_ref[0]
    w2 = w2_ref[...]
    whh = whh_ref[...]
    wout = wout_ref[...]
    bo = bout_ref[0]

    B = x_blk.shape[1]
    onec = jnp.ones((B, 1), f32)
    zc = jnp.zeros((B, 4), f32)

    def lstm_step(xt, h, c, ppcol, upf, bf, af):
        delta = (ppcol - xt[:, 0:1]) * upf
        xu = jnp.concatenate(
            [xt, delta, bf * onec, af * onec, onec, zc], axis=1)
        gates = (jnp.dot(xu, w2, preferred_element_type=f32)
                 + jnp.dot(h, whh, preferred_element_type=f32))
        i_g = gates[:, 0:HIDDEN]
        f_g = gates[:, HIDDEN:2 * HIDDEN]
        g_g = gates[:, 2 * HIDDEN:3 * HIDDEN]
        o_g = gates[:, 3 * HIDDEN:4 * HIDDEN]
        c_n = jax.nn.sigmoid(f_g) * c + jax.nn.sigmoid(i_g) * jnp.tanh(g_g)
        h_n = jax.nn.sigmoid(o_g) * jnp.tanh(c_n)
        outmm = jnp.dot(h_n, wout, preferred_element_type=f32)
        pred = outmm[:, 0:1] + bo
        return h_n, c_n, pred

    h = jnp.where(first, 0.0, h_s[...])
    c = jnp.where(first, 0.0, c_s[...])
    ppcol = jnp.where(first, 0.0, pp_s[...])
    one11 = jnp.ones((1, 1), f32)
    zero11 = jnp.zeros((1, 1), f32)

    @pl.when(pid < WARMUP // CH)
    def _warm():
        hh, cc, pp = h, c, ppcol
        upf = jnp.where(tbase >= WEEK, one11, zero11)
        cols = []
        for r in range(CH):
            hh, cc, pp = lstm_step(x_blk[r], hh, cc, pp, upf, upf, zero11)
            cols.append(pp)
        preds_ref[0] = jnp.concatenate(cols, axis=1)
        h_s[...] = hh
        c_s[...] = cc
        pp_s[...] = pp

    @pl.when(pid >= WARMUP // CH)
    def _zmode():
        hh, cc, pp = h, c, ppcol
        wrow = wrow_s[...]
        wcol = wcol_s[...]
        lane_i = jax.lax.broadcasted_iota(jnp.int32, (1, 256), 1)
        sub_i = jax.lax.broadcasted_iota(jnp.int32, (256, 1), 0)
        lane_w = lane_i[:, :WINDOW]
        sub_w = sub_i[:WINDOW, :]
        tb_mod = jax.lax.rem(tbase - WARMUP, WINDOW)
        pp0 = pp[0:1, 0:1]
        cols = []
        for r in range(CH):
            t = tbase + r
            xt = x_blk[r]
            resid = xt[0:1, 0:1] - pp0
            pg = tb_mod + r
            p = jnp.where(pg >= WINDOW, pg - WINDOW, pg)
            wrow = jnp.where(lane_i == p, resid, wrow)
            wcol = jnp.where(sub_i == p, resid, wcol)
            nv = t - (WARMUP - 1)
            m = jnp.minimum(nv, WINDOW)
            kf = ((m - 1) // 2).astype(f32)
            wr = jnp.where(lane_w < m, wrow[:, :WINDOW], BIG)
            wc = jnp.where(sub_w < m, wcol[:WINDOW, :], BIG)
            cnt = jnp.sum(jnp.where(wc < wr, 1.0, 0.0), axis=0, keepdims=True)
            med = jnp.max(jnp.where(cnt <= kf, wr, -BIG), axis=1, keepdims=True)
            dr = jnp.abs(wr - med)
            dc = jnp.abs(wc - med)
            cnt2 = jnp.sum(jnp.where(dc < dr, 1.0, 0.0), axis=0, keepdims=True)
            mad = jnp.max(jnp.where(cnt2 <= kf, dr, -BIG), axis=1, keepdims=True)
            valid = (nv >= MIN_REQ) & (mad != 0.0)
            denom = jnp.where(valid, mad * MAD_K, 1.0)
            z = jnp.where(valid, (resid - med) / denom, 0.0)
            anom = valid & (jnp.abs(z) > THRESH)
            af = jnp.where(anom, one11, zero11)
            hh, cc, pp = lstm_step(xt, hh, cc, pp, af, af, af)
            pp0 = pp[0:1, 0:1]
            cols.append(pp)
        preds_ref[0] = jnp.concatenate(cols, axis=1)
        h_s[...] = hh
        c_s[...] = cc
        pp_s[...] = pp
        wrow_s[...] = wrow
        wcol_s[...] = wcol

    @pl.when(pid == n_chunks - 1)
    def _final():
        hout_ref[...] = h_s[...]
        cout_ref[...] = c_s[...]


def kernel(x, W_ih, W_hh, b_ih, b_hh, W_out, b_out):
    B, T, F = x.shape
    f32 = jnp.float32
    nc = T // CH
    xr = jnp.transpose(x, (1, 0, 2)).reshape(nc, CH, B, F)
    bias = b_ih + b_hh
    zrow = jnp.zeros_like(bias)
    w2 = jnp.stack(
        [W_ih[:, j] for j in range(F)]
        + [W_ih[:, 0], W_ih[:, F], W_ih[:, F + 1], bias, zrow, zrow, zrow, zrow],
        axis=0)
    whh = jnp.transpose(W_hh)
    wout = jnp.zeros((HIDDEN, 128), f32).at[:, 0].set(W_out[0])
    bo = b_out.astype(f32)

    preds, h, c = pl.pallas_call(
        _fwd_kernel,
        grid=(nc,),
        in_specs=[
            pl.BlockSpec((1, CH, B, F), lambda i: (i, 0, 0, 0)),
            pl.BlockSpec((24, G4), lambda i: (0, 0)),
            pl.BlockSpec((HIDDEN, G4), lambda i: (0, 0)),
            pl.BlockSpec((HIDDEN, 128), lambda i: (0, 0)),
            pl.BlockSpec(memory_space=pltpu.SMEM),
        ],
        out_specs=[
            pl.BlockSpec((1, B, CH), lambda i: (i, 0, 0)),
            pl.BlockSpec((B, HIDDEN), lambda i: (0, 0)),
            pl.BlockSpec((B, HIDDEN), lambda i: (0, 0)),
        ],
        out_shape=[
            jax.ShapeDtypeStruct((nc, B, CH), f32),
            jax.ShapeDtypeStruct((B, HIDDEN), f32),
            jax.ShapeDtypeStruct((B, HIDDEN), f32),
        ],
        scratch_shapes=[
            pltpu.VMEM((B, HIDDEN), f32),
            pltpu.VMEM((B, HIDDEN), f32),
            pltpu.VMEM((B, 1), f32),
            pltpu.VMEM((1, 256), f32),
            pltpu.VMEM((256, 1), f32),
        ],
        compiler_params=pltpu.CompilerParams(
            dimension_semantics=("arbitrary",),
        ),
        name="alt_forecast_lstm",
    )(xr, w2, whh, wout, bo)

    outputs = jnp.transpose(preds, (1, 0, 2)).reshape(B, T, 1)
    return outputs, h, c
